# baseline (speedup 1.0000x reference)
import functools

import jax
import jax.numpy as jnp
from jax import lax
from jax.experimental import pallas as pl
from jax.experimental.pallas import tpu as pltpu
from jax.experimental.pallas import tpu_sc as plsc

N = 10000
E = 320000
D = 128
G = 64
NCONV = 3

BN = 1000
NB = N // BN

NSC = 2
NTILE = 16
NW = NSC * NTILE
EPT = E // NW
CH = 40
NCHUNK = EPT // CH
SUBROWS = 624
REM_OFF = SUBROWS * NTILE
REM = N - REM_OFF
CG = D // 16


def _tables(x, wgs_ref, wgt_ref, wcs_ref, wct_ref, ts_ref, td_ref):
    def t(w_ref):
        p = jnp.dot(x, w_ref[...], preferred_element_type=jnp.float32,
                    precision=lax.Precision.HIGHEST)
        return jnp.exp(jnp.clip(p, -60.0, 60.0))

    ts_ref[:, :D] = t(wgs_ref)
    ts_ref[:, D:] = t(wcs_ref)
    td_ref[:, :D] = t(wgt_ref)
    td_ref[:, D:] = t(wct_ref)


def _embed_body(nodes_ref, emb_ref, wgs, wgt, wcs, wct, x_ref, ts_ref, td_ref):
    noderow = nodes_ref[...].reshape(1, BN)
    feat = lax.broadcasted_iota(jnp.int32, (D, BN), 0)
    ohT = (feat == noderow).astype(jnp.float32)
    x = lax.dot_general(ohT, emb_ref[...], (((0,), (0,)), ((), ())),
                        preferred_element_type=jnp.float32, precision=lax.Precision.HIGHEST)
    x_ref[...] = x
    _tables(x, wgs, wgt, wcs, wct, ts_ref, td_ref)


def _update_body(x_ref, a0_ref, a1_ref, wgs, wgt, wcs, wct, xn_ref, ts_ref, td_ref):
    x = jnp.tanh(x_ref[...] + a0_ref[...] + a1_ref[...])
    xn_ref[...] = x
    _tables(x, wgs, wgt, wcs, wct, ts_ref, td_ref)


def _final_body(x_ref, a0_ref, a1_ref, gi_ref, wpool, counts_ref, wgp, bgp,
                wfc1, bfc1, wfc2, bfc2, wreg, breg, out_ref, acc):
    i = pl.program_id(0)
    x = jnp.tanh(x_ref[...] + a0_ref[...] + a1_ref[...])
    xp = jnp.dot(x, wpool[...], preferred_element_type=jnp.float32, precision=lax.Precision.HIGHEST)
    girow = gi_ref[...].reshape(1, BN)
    seg = lax.broadcasted_iota(jnp.int32, (G, BN), 0)
    ohT = (seg == girow).astype(jnp.float32)
    part = jnp.dot(ohT, xp, preferred_element_type=jnp.float32, precision=lax.Precision.HIGHEST)

    @pl.when(i == 0)
    def _():
        acc[...] = jnp.zeros_like(acc)

    acc[...] += part

    @pl.when(i == NB - 1)
    def _():
        pooled = acc[...] / counts_ref[...]
        g = jnp.maximum(
            jnp.dot(pooled, wgp[...], preferred_element_type=jnp.float32, precision=lax.Precision.HIGHEST) + bgp[...], 0.0)
        g = jnp.maximum(
            jnp.dot(g, wfc1[...], preferred_element_type=jnp.float32, precision=lax.Precision.HIGHEST) + bfc1[...], 0.0)
        g = jnp.maximum(
            jnp.dot(g, wfc2[...], preferred_element_type=jnp.float32, precision=lax.Precision.HIGHEST) + bfc2[...], 0.0)
        out_ref[...] = jnp.dot(g, wreg[...], preferred_element_type=jnp.float32, precision=lax.Precision.HIGHEST) + breg[...]


def _row_spec(w=D):
    return pl.BlockSpec((BN, w), lambda i: (i, 0))


def _full_spec(r, c):
    return pl.BlockSpec((r, c), lambda i: (0, 0))


def _tc_embed(nodes3, embp, wgs, wgt, wcs, wct):
    return pl.pallas_call(
        _embed_body,
        grid=(NB,),
        in_specs=[
            pl.BlockSpec((1, 1, BN), lambda i: (i, 0, 0)),
            _full_spec(D, D), _full_spec(D, D), _full_spec(D, D),
            _full_spec(D, D), _full_spec(D, D),
        ],
        out_specs=[_row_spec(), _row_spec(2 * D), _row_spec(2 * D)],
        out_shape=[
            jax.ShapeDtypeStruct((N, D), jnp.float32),
            jax.ShapeDtypeStruct((N, 2 * D), jnp.float32),
            jax.ShapeDtypeStruct((N, 2 * D), jnp.float32),
        ],
    )(nodes3, embp, wgs, wgt, wcs, wct)


def _tc_update(x, a0, a1, wgs, wgt, wcs, wct):
    return pl.pallas_call(
        _update_body,
        grid=(NB,),
        in_specs=[
            _row_spec(), _row_spec(), _row_spec(),
            _full_spec(D, D), _full_spec(D, D), _full_spec(D, D), _full_spec(D, D),
        ],
        out_specs=[_row_spec(), _row_spec(2 * D), _row_spec(2 * D)],
        out_shape=[
            jax.ShapeDtypeStruct((N, D), jnp.float32),
            jax.ShapeDtypeStruct((N, 2 * D), jnp.float32),
            jax.ShapeDtypeStruct((N, 2 * D), jnp.float32),
        ],
    )(x, a0, a1, wgs, wgt, wcs, wct)


def _tc_final(x, a0, a1, gi3, wpool, counts, wgp, bgp, wfc1, bfc1, wfc2, bfc2,
              wreg, breg):
    return pl.pallas_call(
        _final_body,
        grid=(NB,),
        in_specs=[
            _row_spec(), _row_spec(), _row_spec(),
            pl.BlockSpec((1, 1, BN), lambda i: (i, 0, 0)),
            _full_spec(D, D),
            _full_spec(G, 1),
            _full_spec(D, D), _full_spec(1, D),
            _full_spec(D, D), _full_spec(1, D),
            _full_spec(D, D), _full_spec(1, D),
            _full_spec(D, 1), _full_spec(1, 1),
        ],
        out_specs=pl.BlockSpec((G, 1), lambda i: (0, 0)),
        out_shape=jax.ShapeDtypeStruct((G, 1), jnp.float32),
        scratch_shapes=[pltpu.VMEM((G, D), jnp.float32)],
    )(x, a0, a1, gi3, wpool, counts, wgp, bgp, wfc1, bfc1, wfc2, bfc2, wreg, breg)


def _maybe_when(cond):
    if cond is True:
        return lambda fn: fn()
    return pl.when(cond)


def _gate_compute(rows_s, rows_d, msg_v):
    @plsc.parallel_loop(0, CH, unroll=2)
    def _rows(r):
        for k in range(CG):
            c0 = k * 16
            ea = rows_s[r, pl.ds(c0, 16)] * rows_d[r, pl.ds(c0, 16)]
            eb = jnp.minimum(
                rows_s[r, pl.ds(D + c0, 16)] * rows_d[r, pl.ds(D + c0, 16)],
                1e30)
            den = (1.0 + ea) * (1.0 + eb)
            rec = 1.0 / den
            rec = rec * (2.0 - den * rec)
            msg_v[r, pl.ds(c0, 16)] = (1.0 - eb) * rec


def _sc_conv_body(ts_hbm, td_hbm, esrc_hbm, edst_hbm, zeros_hbm, out_hbm,
                  idx_s0, idx_d0, idx_s1, idx_d1, sid0, sid1,
                  rows_s0, rows_d0, rows_s1, rows_d1,
                  msg_v, agg_sh, sem_g0, sem_g1, sem_i0, sem_i1):
    c = lax.axis_index("c")
    s = lax.axis_index("s")
    wid = c * NTILE + s

    pltpu.sync_copy(zeros_hbm.at[pl.ds(s * SUBROWS, SUBROWS)],
                    agg_sh.at[pl.ds(s * SUBROWS, SUBROWS)])

    @pl.when(s == 0)
    def _():
        pltpu.sync_copy(zeros_hbm.at[pl.ds(REM_OFF, REM)],
                        agg_sh.at[pl.ds(REM_OFF, REM)])

    plsc.subcore_barrier()

    ebase = wid * EPT

    pltpu.sync_copy(esrc_hbm.at[pl.ds(ebase, CH)], idx_s0)
    pltpu.sync_copy(edst_hbm.at[pl.ds(ebase, CH)], idx_d0)
    pltpu.async_copy(ts_hbm.at[idx_s0], rows_s0, sem_g0)
    pltpu.async_copy(td_hbm.at[idx_d0], rows_d0, sem_g0)
    pltpu.async_copy(esrc_hbm.at[pl.ds(ebase + CH, CH)], idx_s1, sem_i1)
    pltpu.async_copy(edst_hbm.at[pl.ds(ebase + CH, CH)], idx_d1, sem_i1)

    def _half(i, idx_sA, idx_dA, sidA, rows_sA, rows_dA, sem_gA, sem_iA,
              idx_sB, idx_dB, rows_sB, rows_dB, sem_gB, sem_iB,
              fire_next_gather, fire_next_idx):
        pltpu.make_async_copy(ts_hbm.at[idx_sA], rows_sA, sem_gA).wait()
        pltpu.make_async_copy(td_hbm.at[idx_dA], rows_dA, sem_gA).wait()

        @_maybe_when(fire_next_gather)
        def _():
            off1 = ebase + (i + 1) * CH
            pltpu.make_async_copy(esrc_hbm.at[pl.ds(off1, CH)], idx_sB, sem_iB).wait()
            pltpu.make_async_copy(edst_hbm.at[pl.ds(off1, CH)], idx_dB, sem_iB).wait()
            pltpu.async_copy(ts_hbm.at[idx_sB], rows_sB, sem_gB)
            pltpu.async_copy(td_hbm.at[idx_dB], rows_dB, sem_gB)

        for t in range(0, CH - 15, 8):
            sidA[pl.ds(t, 16)] = idx_dA[pl.ds(t, 16)]

        _gate_compute(rows_sA, rows_dA, msg_v)
        pltpu.sync_copy(msg_v, agg_sh.at[sidA], add=True)

        @_maybe_when(fire_next_idx)
        def _():
            off2 = ebase + (i + 2) * CH
            pltpu.async_copy(esrc_hbm.at[pl.ds(off2, CH)], idx_sA, sem_iA)
            pltpu.async_copy(edst_hbm.at[pl.ds(off2, CH)], idx_dA, sem_iA)

    def chunk_pair(j, carry):
        i0 = 2 * j
        not_last = j < (NCHUNK // 2 - 1)
        _half(i0, idx_s0, idx_d0, sid0, rows_s0, rows_d0, sem_g0, sem_i0,
              idx_s1, idx_d1, rows_s1, rows_d1, sem_g1, sem_i1,
              True, not_last)
        _half(i0 + 1, idx_s1, idx_d1, sid1, rows_s1, rows_d1, sem_g1, sem_i1,
              idx_s0, idx_d0, rows_s0, rows_d0, sem_g0, sem_i0,
              not_last, not_last)
        return carry

    lax.fori_loop(0, NCHUNK // 2, chunk_pair, 0)
    plsc.subcore_barrier()
    pltpu.sync_copy(agg_sh.at[pl.ds(s * SUBROWS, SUBROWS)],
                    out_hbm.at[pl.ds(c * N + s * SUBROWS, SUBROWS)])

    @pl.when(s == 0)
    def _():
        pltpu.sync_copy(agg_sh.at[pl.ds(REM_OFF, REM)],
                        out_hbm.at[pl.ds(c * N + REM_OFF, REM)])


_sc_conv = functools.partial(
    pl.kernel,
    mesh=plsc.VectorSubcoreMesh(core_axis_name="c", subcore_axis_name="s"),
    out_type=jax.ShapeDtypeStruct((2 * N, D), jnp.float32),
    scratch_types=[
        pltpu.VMEM((CH,), jnp.int32),
        pltpu.VMEM((CH,), jnp.int32),
        pltpu.VMEM((CH,), jnp.int32),
        pltpu.VMEM((CH,), jnp.int32),
        pltpu.VMEM((CH,), jnp.int32),
        pltpu.VMEM((CH,), jnp.int32),
        pltpu.VMEM((CH, 2 * D), jnp.float32),
        pltpu.VMEM((CH, 2 * D), jnp.float32),
        pltpu.VMEM((CH, 2 * D), jnp.float32),
        pltpu.VMEM((CH, 2 * D), jnp.float32),
        pltpu.VMEM((CH, D), jnp.float32),
        pltpu.VMEM_SHARED((N, D), jnp.float32),
        pltpu.SemaphoreType.DMA,
        pltpu.SemaphoreType.DMA,
        pltpu.SemaphoreType.DMA,
        pltpu.SemaphoreType.DMA,
    ],
)(_sc_conv_body)


def kernel(nodes, edge_sources, edge_targets, graph_indices, node_counts,
           emb, Wg, Wc, Wpool, Wgp, bgp, Wfc1, bfc1, Wfc2, bfc2, Wreg, breg):
    f32 = jnp.float32
    nodes3 = nodes.astype(jnp.int32).reshape(NB, 1, BN)
    gi3 = graph_indices.astype(jnp.int32).reshape(NB, 1, BN)
    esrc = edge_sources.astype(jnp.int32)
    edst = edge_targets.astype(jnp.int32)
    counts = node_counts.astype(f32).reshape(G, 1)
    embp = jnp.zeros((D, D), f32).at[: emb.shape[0]].set(emb.astype(f32))
    zeros = jnp.zeros((N, D), f32)
    bgp2 = bgp.reshape(1, D)
    bfc1_2 = bfc1.reshape(1, D)
    bfc2_2 = bfc2.reshape(1, D)
    breg2 = breg.reshape(1, 1)

    Wgn = -Wg
    Wcn = -2.0 * Wc

    x, ts, td = _tc_embed(nodes3, embp, Wgn[0, :D], Wgn[0, D:], Wcn[0, :D], Wcn[0, D:])
    a0 = a1 = None
    for i in range(NCONV):
        agg = _sc_conv(ts, td, esrc, edst, zeros)
        a0, a1 = agg[:N], agg[N:]
        if i + 1 < NCONV:
            x, ts, td = _tc_update(x, a0, a1,
                                   Wgn[i + 1, :D], Wgn[i + 1, D:],
                                   Wcn[i + 1, :D], Wcn[i + 1, D:])
    return _tc_final(x, a0, a1, gi3, Wpool, counts, Wgp, bgp2,
                     Wfc1, bfc1_2, Wfc2, bfc2_2, Wreg, breg2)


# speedup vs baseline: 1.3095x; 1.3095x over previous
import functools

import jax
import jax.numpy as jnp
from jax import lax
from jax.experimental import pallas as pl
from jax.experimental.pallas import tpu as pltpu
from jax.experimental.pallas import tpu_sc as plsc

N = 10000
E = 320000
D = 128
G = 64
NCONV = 3

BN = 1000
NB = N // BN

NSC = 2
NTILE = 16
NW = NSC * NTILE
EPT = E // NW
CH = 40
NCHUNK = EPT // CH
SUBROWS = 624
REM_OFF = SUBROWS * NTILE
REM = N - REM_OFF
CG = D // 16


def _tables(x, wgs_ref, wgt_ref, wcs_ref, wct_ref, ts_ref, td_ref):
    def t(w_ref):
        p = jnp.dot(x, w_ref[...], preferred_element_type=jnp.float32)
        return jnp.exp(jnp.clip(p, -60.0, 60.0))

    ts_ref[:, :D] = t(wgs_ref)
    ts_ref[:, D:] = t(wcs_ref)
    td_ref[:, :D] = t(wgt_ref)
    td_ref[:, D:] = t(wct_ref)


def _embed_body(nodes_ref, emb_ref, wgs, wgt, wcs, wct, x_ref, ts_ref, td_ref):
    noderow = nodes_ref[...].reshape(1, BN)
    feat = lax.broadcasted_iota(jnp.int32, (D, BN), 0)
    ohT = (feat == noderow).astype(jnp.float32)
    x = lax.dot_general(ohT, emb_ref[...], (((0,), (0,)), ((), ())),
                        preferred_element_type=jnp.float32,
                        precision=lax.Precision.HIGHEST)

    x_ref[...] = x
    _tables(x, wgs, wgt, wcs, wct, ts_ref, td_ref)


def _update_body(x_ref, a0_ref, a1_ref, wgs, wgt, wcs, wct, xn_ref, ts_ref, td_ref):
    x = jnp.tanh(x_ref[...] + (a0_ref[...] + a1_ref[...]))
    xn_ref[...] = x
    _tables(x, wgs, wgt, wcs, wct, ts_ref, td_ref)


def _final_body(x_ref, a0_ref, a1_ref, gi_ref, wpool, counts_ref, wgp, bgp,
                wfc1, bfc1, wfc2, bfc2, wreg, breg, out_ref, acc):
    i = pl.program_id(0)
    x = jnp.tanh(x_ref[...] + (a0_ref[...] + a1_ref[...]))
    xp = jnp.dot(x, wpool[...], preferred_element_type=jnp.float32)
    girow = gi_ref[...].reshape(1, BN)
    seg = lax.broadcasted_iota(jnp.int32, (G, BN), 0)
    ohT = (seg == girow).astype(jnp.float32)
    part = jnp.dot(ohT, xp, preferred_element_type=jnp.float32,
                   precision=lax.Precision.HIGHEST)

    @pl.when(i == 0)
    def _():
        acc[...] = jnp.zeros_like(acc)

    acc[...] += part

    @pl.when(i == NB - 1)
    def _():
        pooled = acc[...] / counts_ref[...]
        g = jnp.maximum(
            jnp.dot(pooled, wgp[...], preferred_element_type=jnp.float32) + bgp[...], 0.0)
        g = jnp.maximum(
            jnp.dot(g, wfc1[...], preferred_element_type=jnp.float32) + bfc1[...], 0.0)
        g = jnp.maximum(
            jnp.dot(g, wfc2[...], preferred_element_type=jnp.float32) + bfc2[...], 0.0)
        out_ref[...] = jnp.dot(g, wreg[...], preferred_element_type=jnp.float32) + breg[...]


def _row_spec(w=D):
    return pl.BlockSpec((BN, w), lambda i: (i, 0))


def _full_spec(r, c):
    return pl.BlockSpec((r, c), lambda i: (0, 0))


def _tc_embed(nodes3, embp, wgs, wgt, wcs, wct):
    return pl.pallas_call(
        _embed_body,
        grid=(NB,),
        in_specs=[
            pl.BlockSpec((1, 1, BN), lambda i: (i, 0, 0)),
            _full_spec(D, D), _full_spec(D, D), _full_spec(D, D),
            _full_spec(D, D), _full_spec(D, D),
        ],
        out_specs=[_row_spec(), _row_spec(2 * D), _row_spec(2 * D)],
        out_shape=[
            jax.ShapeDtypeStruct((N, D), jnp.float32),
            jax.ShapeDtypeStruct((N, 2 * D), jnp.float32),
            jax.ShapeDtypeStruct((N, 2 * D), jnp.float32),
        ],
    )(nodes3, embp, wgs, wgt, wcs, wct)


def _tc_update(x, a0, a1, wgs, wgt, wcs, wct):
    return pl.pallas_call(
        _update_body,
        grid=(NB,),
        in_specs=[
            _row_spec(), _row_spec(), _row_spec(),
            _full_spec(D, D), _full_spec(D, D), _full_spec(D, D), _full_spec(D, D),
        ],
        out_specs=[_row_spec(), _row_spec(2 * D), _row_spec(2 * D)],
        out_shape=[
            jax.ShapeDtypeStruct((N, D), jnp.float32),
            jax.ShapeDtypeStruct((N, 2 * D), jnp.float32),
            jax.ShapeDtypeStruct((N, 2 * D), jnp.float32),
        ],
    )(x, a0, a1, wgs, wgt, wcs, wct)


def _tc_final(x, a0, a1, gi3, wpool, counts, wgp, bgp, wfc1, bfc1, wfc2, bfc2,
              wreg, breg):
    return pl.pallas_call(
        _final_body,
        grid=(NB,),
        in_specs=[
            _row_spec(), _row_spec(), _row_spec(),
            pl.BlockSpec((1, 1, BN), lambda i: (i, 0, 0)),
            _full_spec(D, D),
            _full_spec(G, 1),
            _full_spec(D, D), _full_spec(1, D),
            _full_spec(D, D), _full_spec(1, D),
            _full_spec(D, D), _full_spec(1, D),
            _full_spec(D, 1), _full_spec(1, 1),
        ],
        out_specs=pl.BlockSpec((G, 1), lambda i: (0, 0)),
        out_shape=jax.ShapeDtypeStruct((G, 1), jnp.float32),
        scratch_shapes=[pltpu.VMEM((G, D), jnp.float32)],
    )(x, a0, a1, gi3, wpool, counts, wgp, bgp, wfc1, bfc1, wfc2, bfc2, wreg, breg)


def _maybe_when(cond):
    if cond is True:
        return lambda fn: fn()
    return pl.when(cond)


def _gate_compute(rows_s, rows_d, msg_v):
    @plsc.parallel_loop(0, CH, unroll=2)
    def _rows(r):
        for k in range(CG):
            c0 = k * 16
            ea = rows_s[r, pl.ds(c0, 16)] * rows_d[r, pl.ds(c0, 16)]
            eb = jnp.minimum(
                rows_s[r, pl.ds(D + c0, 16)] * rows_d[r, pl.ds(D + c0, 16)],
                1e30)
            msg_v[r, pl.ds(c0, 16)] = (1.0 - eb) / ((1.0 + ea) * (1.0 + eb))


def _sc_conv_body(ts_hbm, td_hbm, esrc_hbm, edst_hbm, zeros_hbm, out_hbm,
                  idx_s0, idx_d0, idx_s1, idx_d1, sid0, sid1,
                  rows_s0, rows_d0, rows_s1, rows_d1,
                  msg_v, agg_sh, sem_g0, sem_g1, sem_i0, sem_i1):
    c = lax.axis_index("c")
    s = lax.axis_index("s")
    wid = c * NTILE + s

    pltpu.sync_copy(zeros_hbm.at[pl.ds(s * SUBROWS, SUBROWS)],
                    agg_sh.at[pl.ds(s * SUBROWS, SUBROWS)])

    @pl.when(s == 0)
    def _():
        pltpu.sync_copy(zeros_hbm.at[pl.ds(REM_OFF, REM)],
                        agg_sh.at[pl.ds(REM_OFF, REM)])

    plsc.subcore_barrier()

    ebase = wid * EPT

    pltpu.sync_copy(esrc_hbm.at[pl.ds(ebase, CH)], idx_s0)
    pltpu.sync_copy(edst_hbm.at[pl.ds(ebase, CH)], idx_d0)
    pltpu.async_copy(ts_hbm.at[idx_s0], rows_s0, sem_g0)
    pltpu.async_copy(td_hbm.at[idx_d0], rows_d0, sem_g0)
    pltpu.async_copy(esrc_hbm.at[pl.ds(ebase + CH, CH)], idx_s1, sem_i1)
    pltpu.async_copy(edst_hbm.at[pl.ds(ebase + CH, CH)], idx_d1, sem_i1)

    def _half(i, idx_sA, idx_dA, sidA, rows_sA, rows_dA, sem_gA, sem_iA,
              idx_sB, idx_dB, rows_sB, rows_dB, sem_gB, sem_iB,
              fire_next_gather, fire_next_idx):
        pltpu.make_async_copy(ts_hbm.at[idx_sA], rows_sA, sem_gA).wait()
        pltpu.make_async_copy(td_hbm.at[idx_dA], rows_dA, sem_gA).wait()

        @_maybe_when(fire_next_gather)
        def _():
            off1 = ebase + (i + 1) * CH
            pltpu.make_async_copy(esrc_hbm.at[pl.ds(off1, CH)], idx_sB, sem_iB).wait()
            pltpu.make_async_copy(edst_hbm.at[pl.ds(off1, CH)], idx_dB, sem_iB).wait()
            pltpu.async_copy(ts_hbm.at[idx_sB], rows_sB, sem_gB)
            pltpu.async_copy(td_hbm.at[idx_dB], rows_dB, sem_gB)

        for t in range(0, CH - 15, 8):
            sidA[pl.ds(t, 16)] = idx_dA[pl.ds(t, 16)]

        _gate_compute(rows_sA, rows_dA, msg_v)
        pltpu.sync_copy(msg_v, agg_sh.at[sidA], add=True)

        @_maybe_when(fire_next_idx)
        def _():
            off2 = ebase + (i + 2) * CH
            pltpu.async_copy(esrc_hbm.at[pl.ds(off2, CH)], idx_sA, sem_iA)
            pltpu.async_copy(edst_hbm.at[pl.ds(off2, CH)], idx_dA, sem_iA)

    def chunk_pair(j, carry):
        i0 = 2 * j
        not_last = j < (NCHUNK // 2 - 1)
        _half(i0, idx_s0, idx_d0, sid0, rows_s0, rows_d0, sem_g0, sem_i0,
              idx_s1, idx_d1, rows_s1, rows_d1, sem_g1, sem_i1,
              True, not_last)
        _half(i0 + 1, idx_s1, idx_d1, sid1, rows_s1, rows_d1, sem_g1, sem_i1,
              idx_s0, idx_d0, rows_s0, rows_d0, sem_g0, sem_i0,
              not_last, not_last)
        return carry

    lax.fori_loop(0, NCHUNK // 2, chunk_pair, 0)
    plsc.subcore_barrier()
    pltpu.sync_copy(agg_sh.at[pl.ds(s * SUBROWS, SUBROWS)],
                    out_hbm.at[pl.ds(c * N + s * SUBROWS, SUBROWS)])

    @pl.when(s == 0)
    def _():
        pltpu.sync_copy(agg_sh.at[pl.ds(REM_OFF, REM)],
                        out_hbm.at[pl.ds(c * N + REM_OFF, REM)])


_sc_conv = functools.partial(
    pl.kernel,
    mesh=plsc.VectorSubcoreMesh(core_axis_name="c", subcore_axis_name="s"),
    out_type=jax.ShapeDtypeStruct((2 * N, D), jnp.float32),
    scratch_types=[
        pltpu.VMEM((CH,), jnp.int32),
        pltpu.VMEM((CH,), jnp.int32),
        pltpu.VMEM((CH,), jnp.int32),
        pltpu.VMEM((CH,), jnp.int32),
        pltpu.VMEM((CH,), jnp.int32),
        pltpu.VMEM((CH,), jnp.int32),
        pltpu.VMEM((CH, 2 * D), jnp.float32),
        pltpu.VMEM((CH, 2 * D), jnp.float32),
        pltpu.VMEM((CH, 2 * D), jnp.float32),
        pltpu.VMEM((CH, 2 * D), jnp.float32),
        pltpu.VMEM((CH, D), jnp.float32),
        pltpu.VMEM_SHARED((N, D), jnp.float32),
        pltpu.SemaphoreType.DMA,
        pltpu.SemaphoreType.DMA,
        pltpu.SemaphoreType.DMA,
        pltpu.SemaphoreType.DMA,
    ],
)(_sc_conv_body)


def kernel(nodes, edge_sources, edge_targets, graph_indices, node_counts,
           emb, Wg, Wc, Wpool, Wgp, bgp, Wfc1, bfc1, Wfc2, bfc2, Wreg, breg):
    f32 = jnp.float32
    nodes3 = nodes.astype(jnp.int32).reshape(NB, 1, BN)
    gi3 = graph_indices.astype(jnp.int32).reshape(NB, 1, BN)
    esrc = edge_sources.astype(jnp.int32)
    edst = edge_targets.astype(jnp.int32)
    counts = node_counts.astype(f32).reshape(G, 1)
    embp = jnp.zeros((D, D), f32).at[: emb.shape[0]].set(emb.astype(f32))
    zeros = jnp.zeros((N, D), f32)
    bgp2 = bgp.reshape(1, D)
    bfc1_2 = bfc1.reshape(1, D)
    bfc2_2 = bfc2.reshape(1, D)
    breg2 = breg.reshape(1, 1)

    Wgn = -Wg
    Wcn = -2.0 * Wc

    x, ts, td = _tc_embed(nodes3, embp, Wgn[0, :D], Wgn[0, D:], Wcn[0, :D], Wcn[0, D:])
    a0 = a1 = None
    for i in range(NCONV):
        agg = _sc_conv(ts, td, esrc, edst, zeros)
        a0, a1 = agg[:N], agg[N:]
        if i + 1 < NCONV:
            x, ts, td = _tc_update(x, a0, a1,
                                   Wgn[i + 1, :D], Wgn[i + 1, D:],
                                   Wcn[i + 1, :D], Wcn[i + 1, D:])
    return _tc_final(x, a0, a1, gi3, Wpool, counts, Wgp, bgp2,
                     Wfc1, bfc1_2, Wfc2, bfc2_2, Wreg, breg2)
